# baseline (speedup 1.0000x reference)
import functools

import jax
import jax.numpy as jnp
from jax.experimental import pallas as pl
from jax.experimental.pallas import tpu as pltpu
from jax.experimental.pallas import tpu_sc as plsc


def _ceil_to(x, m):
  return ((x + m - 1) // m) * m


def _sc_gather_pair(A3, srcf, dstf):
  epad, = srcf.shape
  _, sl, lanes = A3.shape
  nw = 32
  rows_w = epad // nw
  b = 8
  nb = rows_w // b
  mesh = plsc.VectorSubcoreMesh(core_axis_name="c", subcore_axis_name="s")

  @functools.partial(
      pl.kernel, mesh=mesh,
      out_type=[jax.ShapeDtypeStruct((epad, sl, lanes), A3.dtype)] * 2,
      scratch_types=[
          pltpu.VMEM((rows_w,), jnp.int32),
          pltpu.VMEM((rows_w,), jnp.int32),
          pltpu.VMEM((b, sl, lanes), A3.dtype),
          pltpu.VMEM((b, sl, lanes), A3.dtype),
          pltpu.SemaphoreType.DMA,
          pltpu.SemaphoreType.DMA,
      ])
  def gk(a_hbm, s_hbm, d_hbm, os_hbm, od_hbm, si, di, sb, db, sem1, sem2):
    wid = jax.lax.axis_index("s") * 2 + jax.lax.axis_index("c")
    base = wid * rows_w
    pltpu.sync_copy(s_hbm.at[pl.ds(base, rows_w)], si)
    pltpu.sync_copy(d_hbm.at[pl.ds(base, rows_w)], di)

    def body(t, carry):
      off = t * b
      cp1 = pltpu.async_copy(a_hbm.at[si.at[pl.ds(off, b)]], sb, sem1)
      cp2 = pltpu.async_copy(a_hbm.at[di.at[pl.ds(off, b)]], db, sem2)
      cp1.wait()
      pltpu.sync_copy(sb, os_hbm.at[pl.ds(base + off, b)])
      cp2.wait()
      pltpu.sync_copy(db, od_hbm.at[pl.ds(base + off, b)])
      return carry

    jax.lax.fori_loop(0, nb, body, 0)

  return gk(A3, srcf, dstf)


def _vmul_body(a_ref, b_ref, o_ref):
  o_ref[...] = a_ref[...] & b_ref[...]


def _vmul_call(a, b, rb, cb):
  n, w = a.shape
  return pl.pallas_call(
      _vmul_body,
      grid=(n // rb, w // cb),
      in_specs=[
          pl.BlockSpec((rb, cb), lambda i, j: (i, j)),
          pl.BlockSpec((rb, cb), lambda i, j: (i, j)),
      ],
      out_specs=pl.BlockSpec((rb, cb), lambda i, j: (i, j)),
      out_shape=jax.ShapeDtypeStruct((n, w), jnp.int32),
  )(a, b)


def _quad_body(nk, v_ref, a_ref, quad_ref, rs_ref, acc_ref):
  j = pl.program_id(1)
  k = pl.program_id(2)
  bk = a_ref.shape[0]
  bj = a_ref.shape[1]

  @pl.when(k == 0)
  def _():
    acc_ref[...] = jnp.zeros_like(acc_ref)

  vk = v_ref[:, pl.ds(k * bk, bk)]
  acc_ref[...] += jax.lax.dot_general(
      vk, a_ref[...], (((1,), (0,)), ((), ())),
      preferred_element_type=jnp.int32)

  @pl.when(jnp.logical_and(j == 0, k == 0))
  def _():
    rs_ref[...] = jnp.zeros_like(rs_ref)

  @pl.when(j == 0)
  def _():
    rs_ref[0, 0, :] += jnp.sum(vk.astype(jnp.int32), axis=1)

  @pl.when(k == nk - 1)
  def _():
    vj = v_ref[:, pl.ds(j * bj, bj)].astype(jnp.int32)
    part = jnp.sum(acc_ref[...] * vj, axis=1)

    @pl.when(j == 0)
    def _():
      quad_ref[0, 0, :] = part

    @pl.when(j > 0)
    def _():
      quad_ref[0, 0, :] += part


def _quad_call(V, A, nchunk, c, bk, bj):
  np_ = A.shape[0]
  nj = np_ // bj
  nk = np_ // bk
  return pl.pallas_call(
      functools.partial(_quad_body, nk),
      grid=(nchunk, nj, nk),
      in_specs=[
          pl.BlockSpec((c, np_), lambda ci, j, k: (ci, 0)),
          pl.BlockSpec((bk, bj), lambda ci, j, k: (k, j)),
      ],
      out_specs=[
          pl.BlockSpec((1, 1, c), lambda ci, j, k: (ci, 0, 0)),
          pl.BlockSpec((1, 1, c), lambda ci, j, k: (ci, 0, 0)),
      ],
      out_shape=[
          jax.ShapeDtypeStruct((nchunk, 1, c), jnp.int32),
          jax.ShapeDtypeStruct((nchunk, 1, c), jnp.int32),
      ],
      scratch_shapes=[pltpu.VMEM((c, bj), jnp.int32)],
      compiler_params=pltpu.CompilerParams(
          dimension_semantics=("parallel", "arbitrary", "arbitrary")),
  )(V, A)


def _segmax_body(ce, pk_ref, nr_ref, y_ref, m_ref):

  @pl.when(pl.program_id(0) == 0)
  def _():
    m_ref[...] = jnp.full_like(m_ref, -jnp.inf)

  def step(e, carry):
    pkv = pk_ref[0, 0, e]
    s = pkv >> 14
    d = pkv & 16383
    nr = nr_ref[0, 0, e]
    row = y_ref[pl.ds(s, 1), :]
    m_ref[pl.ds(d, 1), :] = jnp.maximum(m_ref[pl.ds(d, 1), :], nr * row)
    return carry

  jax.lax.fori_loop(0, ce, step, 0)


def _segmax_call(pk, nr, y, n_out, nchunk, ce):
  d = y.shape[1]
  return pl.pallas_call(
      functools.partial(_segmax_body, ce),
      grid=(nchunk,),
      in_specs=[
          pl.BlockSpec((1, 1, ce), lambda c: (c, 0, 0),
                       memory_space=pltpu.SMEM),
          pl.BlockSpec((1, 1, ce), lambda c: (c, 0, 0),
                       memory_space=pltpu.SMEM),
          pl.BlockSpec(y.shape, lambda c: (0, 0)),
      ],
      out_specs=pl.BlockSpec((n_out, d), lambda c: (0, 0)),
      out_shape=jax.ShapeDtypeStruct((n_out, d), jnp.float32),
      compiler_params=pltpu.CompilerParams(
          dimension_semantics=("arbitrary",)),
  )(pk, nr, y)


def _lin_body(h_ref, w_ref, o_ref):
  o_ref[...] = jax.lax.dot_general(
      h_ref[...], w_ref[...], (((1,), (1,)), ((), ())),
      preferred_element_type=jnp.float32)


def _lin_call(h, w, rb):
  n, _ = h.shape
  dout = w.shape[0]
  return pl.pallas_call(
      _lin_body,
      grid=(n // rb,),
      in_specs=[
          pl.BlockSpec((rb, h.shape[1]), lambda i: (i, 0)),
          pl.BlockSpec(w.shape, lambda i: (0, 0)),
      ],
      out_specs=pl.BlockSpec((rb, dout), lambda i: (i, 0)),
      out_shape=jax.ShapeDtypeStruct((n, dout), jnp.float32),
  )(h, w)


def _lin_relu_body(h_ref, b_ref, w_ref, o_ref):
  a = jnp.maximum(h_ref[...] + b_ref[...], 0.0)
  o_ref[...] = jax.lax.dot_general(
      a, w_ref[...], (((1,), (1,)), ((), ())),
      preferred_element_type=jnp.float32)


def _lin_relu_call(h, b, w, rb):
  n, _ = h.shape
  dout = w.shape[0]
  return pl.pallas_call(
      _lin_relu_body,
      grid=(n // rb,),
      in_specs=[
          pl.BlockSpec((rb, h.shape[1]), lambda i: (i, 0)),
          pl.BlockSpec((1, h.shape[1]), lambda i: (0, 0)),
          pl.BlockSpec(w.shape, lambda i: (0, 0)),
      ],
      out_specs=pl.BlockSpec((rb, dout), lambda i: (i, 0)),
      out_shape=jax.ShapeDtypeStruct((n, dout), jnp.float32),
  )(h, b, w)


def _final_body(h_ref, b_ref, wf_ref, bf_ref, o_ref):
  a = jnp.maximum(h_ref[...] + b_ref[...], 0.0)
  logits = jax.lax.dot_general(
      a, wf_ref[...], (((1,), (1,)), ((), ())),
      preferred_element_type=jnp.float32) + bf_ref[...]
  mx = jnp.max(logits, axis=-1, keepdims=True)
  lse = mx + jnp.log(jnp.sum(jnp.exp(logits - mx), axis=-1, keepdims=True))
  o_ref[...] = logits - lse


def _final_call(h, b, wf, bf, rb):
  n, _ = h.shape
  ncls = wf.shape[0]
  return pl.pallas_call(
      _final_body,
      grid=(n // rb,),
      in_specs=[
          pl.BlockSpec((rb, h.shape[1]), lambda i: (i, 0)),
          pl.BlockSpec((1, h.shape[1]), lambda i: (0, 0)),
          pl.BlockSpec(wf.shape, lambda i: (0, 0)),
          pl.BlockSpec((1, ncls), lambda i: (0, 0)),
      ],
      out_specs=pl.BlockSpec((rb, ncls), lambda i: (i, 0)),
      out_shape=jax.ShapeDtypeStruct((n, ncls), jnp.float32),
  )(h, b, wf, bf)


def kernel(x, edge_index, W0, b0, W1, b1, Wf, bf):
  n, d_in = x.shape
  e = edge_index.shape[1]
  ef = e + n
  np_ = _ceil_to(n, 512)

  diag = jnp.arange(n, dtype=jnp.int32)
  ei0 = edge_index[0].astype(jnp.int32)
  ei1 = edge_index[1].astype(jnp.int32)

  A = jnp.zeros((np_, np_), dtype=jnp.bfloat16)
  A = A.at[ei0, ei1].set(jnp.bfloat16(1))
  A = A.at[ei1, ei0].set(jnp.bfloat16(1))
  A = A.at[diag, diag].set(jnp.bfloat16(1))
  A_i8 = A.astype(jnp.int8)

  c = 1024
  epad = _ceil_to(ef, c)
  nchunk = epad // c
  zpad = jnp.zeros((epad - ef,), dtype=jnp.int32)
  srcf = jnp.concatenate([ei0, diag, zpad])
  dstf = jnp.concatenate([ei1, diag, zpad])

  half = np_ // 2
  A_pack = jax.lax.bitcast_convert_type(
      A.reshape(np_, half, 2), jnp.int32)
  a_s, a_t = _sc_gather_pair(A_pack.reshape(np_, half // 128, 128),
                             srcf, dstf)
  V_pack = _vmul_call(a_s.reshape(epad, half), a_t.reshape(epad, half),
                      min(256, epad), min(1024, half))
  V = jax.lax.bitcast_convert_type(V_pack, jnp.bfloat16).reshape(
      epad, np_).astype(jnp.int8)

  bk = bj = min(512, np_)
  quad, rs = _quad_call(V, A_i8, nchunk, c, bk, bj)
  quad = quad.reshape(-1)[:ef].astype(jnp.float32)
  rs = rs.reshape(-1)[:ef].astype(jnp.float32)

  inter = 0.5 * (quad + rs)
  ecnt = inter[e:]
  src = srcf[:ef]
  dst = dstf[:ef]
  uni = ecnt[src] + ecnt[dst] - inter
  norm = jnp.where(uni > 0, inter / uni, jnp.float32(0.0))

  ce = 1000 if ef % 1000 == 0 else ef
  echunk = _ceil_to(ef, ce)
  nechunk = echunk // ce
  n_out = n + 8
  pad_e = echunk - ef
  pk = jnp.concatenate([
      (src << 14) | dst,
      jnp.full((pad_e,), n, dtype=jnp.int32),
  ]).reshape(nechunk, 1, ce)
  nrm = jnp.concatenate([norm, jnp.zeros((pad_e,), jnp.float32)])
  nrm = nrm.reshape(nechunk, 1, ce)

  rb = 1000 if n % 1000 == 0 else n

  h = _lin_call(x, W0, rb)
  m = _segmax_call(pk, nrm, h, n_out, nechunk, ce)[:n]
  h = _lin_relu_call(m, b0.reshape(1, -1), W1, rb)
  m = _segmax_call(pk, nrm, h, n_out, nechunk, ce)[:n]
  out = _final_call(m, b1.reshape(1, -1), Wf, bf.reshape(1, -1), rb)
  return out


# speedup vs baseline: 1.3371x; 1.3371x over previous
import functools

import jax
import jax.numpy as jnp
from jax.experimental import pallas as pl
from jax.experimental.pallas import tpu as pltpu
from jax.experimental.pallas import tpu_sc as plsc


def _ceil_to(x, m):
  return ((x + m - 1) // m) * m


def _sc_gather_and(A3, srcf, dstf):
  epad, = srcf.shape
  _, sl, lanes = A3.shape
  nw = 32
  rows_w = epad // nw
  b = 8
  nb = rows_w // b
  mesh = plsc.VectorSubcoreMesh(core_axis_name="c", subcore_axis_name="s")

  @functools.partial(
      pl.kernel, mesh=mesh,
      out_type=jax.ShapeDtypeStruct((epad, sl, lanes), A3.dtype),
      scratch_types=[
          pltpu.VMEM((rows_w,), jnp.int32),
          pltpu.VMEM((rows_w,), jnp.int32),
          pltpu.VMEM((b, sl, lanes), A3.dtype),
          pltpu.VMEM((b, sl, lanes), A3.dtype),
          pltpu.SemaphoreType.DMA,
          pltpu.SemaphoreType.DMA,
      ])
  def gk(a_hbm, s_hbm, d_hbm, o_hbm, si, di, sb, db, sem1, sem2):
    wid = jax.lax.axis_index("s") * 2 + jax.lax.axis_index("c")
    base = wid * rows_w
    pltpu.sync_copy(s_hbm.at[pl.ds(base, rows_w)], si)
    pltpu.sync_copy(d_hbm.at[pl.ds(base, rows_w)], di)

    def body(t, carry):
      off = t * b
      cp1 = pltpu.async_copy(a_hbm.at[si.at[pl.ds(off, b)]], sb, sem1)
      cp2 = pltpu.async_copy(a_hbm.at[di.at[pl.ds(off, b)]], db, sem2)
      cp1.wait()
      cp2.wait()
      for r in range(b):
        def qbody(q, cin):
          for l in range(lanes // 16):
            ls = pl.ds(l * 16, 16)
            sb[r, q, ls] = sb[r, q, ls] & db[r, q, ls]
          return cin
        jax.lax.fori_loop(0, sl, qbody, 0)
      pltpu.sync_copy(sb, o_hbm.at[pl.ds(base + off, b)])
      return carry

    jax.lax.fori_loop(0, nb, body, 0)

  return gk(A3, srcf, dstf)


def _quad_body(nk, adt, v_ref, a_ref, quad_ref, rs_ref, acc_ref):
  j = pl.program_id(1)
  k = pl.program_id(2)
  bk = a_ref.shape[0]
  bj = a_ref.shape[1]

  @pl.when(k == 0)
  def _():
    acc_ref[...] = jnp.zeros_like(acc_ref)

  vk = v_ref[:, pl.ds(k * bk, bk)]
  acc_ref[...] += jax.lax.dot_general(
      vk, a_ref[...], (((1,), (0,)), ((), ())),
      preferred_element_type=adt)

  @pl.when(jnp.logical_and(j == 0, k == 0))
  def _():
    rs_ref[...] = jnp.zeros_like(rs_ref)

  @pl.when(j == 0)
  def _():
    rs_ref[0, 0, :] += jnp.sum(vk.astype(adt), axis=1)

  @pl.when(k == nk - 1)
  def _():
    vj = v_ref[:, pl.ds(j * bj, bj)].astype(adt)
    part = jnp.sum(acc_ref[...] * vj, axis=1)

    @pl.when(j == 0)
    def _():
      quad_ref[0, 0, :] = part

    @pl.when(j > 0)
    def _():
      quad_ref[0, 0, :] += part


def _quad_call(V, A, nchunk, c, bk, bj):
  np_ = A.shape[0]
  nj = np_ // bj
  nk = np_ // bk
  adt = jnp.int32 if V.dtype == jnp.int8 else jnp.float32
  return pl.pallas_call(
      functools.partial(_quad_body, nk, adt),
      grid=(nchunk, nj, nk),
      in_specs=[
          pl.BlockSpec((c, np_), lambda ci, j, k: (ci, 0)),
          pl.BlockSpec((bk, bj), lambda ci, j, k: (k, j)),
      ],
      out_specs=[
          pl.BlockSpec((1, 1, c), lambda ci, j, k: (ci, 0, 0)),
          pl.BlockSpec((1, 1, c), lambda ci, j, k: (ci, 0, 0)),
      ],
      out_shape=[
          jax.ShapeDtypeStruct((nchunk, 1, c), adt),
          jax.ShapeDtypeStruct((nchunk, 1, c), adt),
      ],
      scratch_shapes=[pltpu.VMEM((c, bj), adt)],
      compiler_params=pltpu.CompilerParams(
          dimension_semantics=("parallel", "arbitrary", "arbitrary")),
  )(V, A)


def _segmax_body(ce, pk_ref, nr_ref, y_ref, m_ref):

  @pl.when(pl.program_id(0) == 0)
  def _():
    m_ref[...] = jnp.full_like(m_ref, -jnp.inf)

  def step(e, carry):
    pkv = pk_ref[0, 0, e]
    s = pkv >> 14
    d = pkv & 16383
    nr = nr_ref[0, 0, e]
    row = y_ref[pl.ds(s, 1), :]
    m_ref[pl.ds(d, 1), :] = jnp.maximum(m_ref[pl.ds(d, 1), :], nr * row)
    return carry

  jax.lax.fori_loop(0, ce, step, 0)


def _segmax_call(pk, nr, y, n_out, nchunk, ce):
  d = y.shape[1]
  return pl.pallas_call(
      functools.partial(_segmax_body, ce),
      grid=(nchunk,),
      in_specs=[
          pl.BlockSpec((1, 1, ce), lambda c: (c, 0, 0),
                       memory_space=pltpu.SMEM),
          pl.BlockSpec((1, 1, ce), lambda c: (c, 0, 0),
                       memory_space=pltpu.SMEM),
          pl.BlockSpec(y.shape, lambda c: (0, 0)),
      ],
      out_specs=pl.BlockSpec((n_out, d), lambda c: (0, 0)),
      out_shape=jax.ShapeDtypeStruct((n_out, d), jnp.float32),
      compiler_params=pltpu.CompilerParams(
          dimension_semantics=("arbitrary",)),
  )(pk, nr, y)


def _lin_body(h_ref, w_ref, o_ref):
  o_ref[...] = jax.lax.dot_general(
      h_ref[...], w_ref[...], (((1,), (1,)), ((), ())),
      preferred_element_type=jnp.float32)


def _lin_call(h, w, rb):
  n, _ = h.shape
  dout = w.shape[0]
  return pl.pallas_call(
      _lin_body,
      grid=(n // rb,),
      in_specs=[
          pl.BlockSpec((rb, h.shape[1]), lambda i: (i, 0)),
          pl.BlockSpec(w.shape, lambda i: (0, 0)),
      ],
      out_specs=pl.BlockSpec((rb, dout), lambda i: (i, 0)),
      out_shape=jax.ShapeDtypeStruct((n, dout), jnp.float32),
  )(h, w)


def _lin_relu_body(h_ref, b_ref, w_ref, o_ref):
  a = jnp.maximum(h_ref[...] + b_ref[...], 0.0)
  o_ref[...] = jax.lax.dot_general(
      a, w_ref[...], (((1,), (1,)), ((), ())),
      preferred_element_type=jnp.float32)


def _lin_relu_call(h, b, w, rb):
  n, _ = h.shape
  dout = w.shape[0]
  return pl.pallas_call(
      _lin_relu_body,
      grid=(n // rb,),
      in_specs=[
          pl.BlockSpec((rb, h.shape[1]), lambda i: (i, 0)),
          pl.BlockSpec((1, h.shape[1]), lambda i: (0, 0)),
          pl.BlockSpec(w.shape, lambda i: (0, 0)),
      ],
      out_specs=pl.BlockSpec((rb, dout), lambda i: (i, 0)),
      out_shape=jax.ShapeDtypeStruct((n, dout), jnp.float32),
  )(h, b, w)


def _final_body(h_ref, b_ref, wf_ref, bf_ref, o_ref):
  a = jnp.maximum(h_ref[...] + b_ref[...], 0.0)
  logits = jax.lax.dot_general(
      a, wf_ref[...], (((1,), (1,)), ((), ())),
      preferred_element_type=jnp.float32) + bf_ref[...]
  mx = jnp.max(logits, axis=-1, keepdims=True)
  lse = mx + jnp.log(jnp.sum(jnp.exp(logits - mx), axis=-1, keepdims=True))
  o_ref[...] = logits - lse


def _final_call(h, b, wf, bf, rb):
  n, _ = h.shape
  ncls = wf.shape[0]
  return pl.pallas_call(
      _final_body,
      grid=(n // rb,),
      in_specs=[
          pl.BlockSpec((rb, h.shape[1]), lambda i: (i, 0)),
          pl.BlockSpec((1, h.shape[1]), lambda i: (0, 0)),
          pl.BlockSpec(wf.shape, lambda i: (0, 0)),
          pl.BlockSpec((1, ncls), lambda i: (0, 0)),
      ],
      out_specs=pl.BlockSpec((rb, ncls), lambda i: (i, 0)),
      out_shape=jax.ShapeDtypeStruct((n, ncls), jnp.float32),
  )(h, b, wf, bf)


def kernel(x, edge_index, W0, b0, W1, b1, Wf, bf):
  n, d_in = x.shape
  e = edge_index.shape[1]
  ef = e + n
  np_ = _ceil_to(n, 512)

  diag = jnp.arange(n, dtype=jnp.int32)
  ei0 = edge_index[0].astype(jnp.int32)
  ei1 = edge_index[1].astype(jnp.int32)

  A = jnp.zeros((np_, np_), dtype=jnp.bfloat16)
  A = A.at[ei0, ei1].set(jnp.bfloat16(1))
  A = A.at[ei1, ei0].set(jnp.bfloat16(1))
  A = A.at[diag, diag].set(jnp.bfloat16(1))

  c = 1024
  epad = _ceil_to(ef, c)
  nchunk = epad // c
  zpad = jnp.zeros((epad - ef,), dtype=jnp.int32)
  srcf = jnp.concatenate([ei0, diag, zpad])
  dstf = jnp.concatenate([ei1, diag, zpad])

  half = np_ // 2
  A_pack = jax.lax.bitcast_convert_type(
      A.reshape(np_, half, 2), jnp.int32)
  V_pack = _sc_gather_and(A_pack.reshape(np_, half // 128, 128),
                          srcf, dstf)
  V = jax.lax.bitcast_convert_type(
      V_pack.reshape(epad, half), jnp.bfloat16).reshape(epad, np_)

  bk = bj = min(512, np_)
  quad, rs = _quad_call(V, A, nchunk, c, bk, bj)
  quad = quad.reshape(-1)[:ef].astype(jnp.float32)
  rs = rs.reshape(-1)[:ef].astype(jnp.float32)

  inter = 0.5 * (quad + rs)
  ecnt = inter[e:]
  src = srcf[:ef]
  dst = dstf[:ef]
  uni = ecnt[src] + ecnt[dst] - inter
  norm = jnp.where(uni > 0, inter / uni, jnp.float32(0.0))

  ce = 1000 if ef % 1000 == 0 else ef
  echunk = _ceil_to(ef, ce)
  nechunk = echunk // ce
  n_out = n + 8
  pad_e = echunk - ef
  pk = jnp.concatenate([
      (src << 14) | dst,
      jnp.full((pad_e,), n, dtype=jnp.int32),
  ]).reshape(nechunk, 1, ce)
  nrm = jnp.concatenate([norm, jnp.zeros((pad_e,), jnp.float32)])
  nrm = nrm.reshape(nechunk, 1, ce)

  rb = 1000 if n % 1000 == 0 else n

  h = _lin_call(x, W0, rb)
  m = _segmax_call(pk, nrm, h, n_out, nechunk, ce)[:n]
  h = _lin_relu_call(m, b0.reshape(1, -1), W1, rb)
  m = _segmax_call(pk, nrm, h, n_out, nechunk, ce)[:n]
  out = _final_call(m, b1.reshape(1, -1), Wf, bf.reshape(1, -1), rb)
  return out


# speedup vs baseline: 1.3378x; 1.0005x over previous
import functools

import jax
import jax.numpy as jnp
from jax.experimental import pallas as pl
from jax.experimental.pallas import tpu as pltpu
from jax.experimental.pallas import tpu_sc as plsc


def _ceil_to(x, m):
  return ((x + m - 1) // m) * m


def _sc_gather_and(A3, pkf):
  epad, = pkf.shape
  _, sl, lanes = A3.shape
  nw = 32
  rows_w = epad // nw
  b = 8
  nb = rows_w // b
  mesh = plsc.VectorSubcoreMesh(core_axis_name="c", subcore_axis_name="s")

  @functools.partial(
      pl.kernel, mesh=mesh,
      out_type=jax.ShapeDtypeStruct((epad, sl, lanes), A3.dtype),
      scratch_types=[
          pltpu.VMEM((rows_w + 16,), jnp.int32),
          pltpu.VMEM((16,), jnp.int32),
          pltpu.VMEM((16,), jnp.int32),
          pltpu.VMEM((16,), jnp.int32),
          pltpu.VMEM((b, sl, lanes), A3.dtype),
          pltpu.VMEM((b, sl, lanes), A3.dtype),
          pltpu.VMEM((b, sl, lanes), A3.dtype),
          pltpu.SemaphoreType.DMA,
          pltpu.SemaphoreType.DMA,
      ])
  def gk(a_hbm, pk_hbm, o_hbm, pki, sx0, sx1, dx, sb0, sb1, db, gs, gd):
    wid = jax.lax.axis_index("s") * 2 + jax.lax.axis_index("c")
    base = wid * rows_w
    pltpu.sync_copy(pk_hbm.at[pl.ds(base, rows_w)], pki.at[pl.ds(0, rows_w)])

    def load_idx(t, sx):
      v = pki[pl.ds(t * b, 16)]
      sx[...] = v >> 14
      return v & 16383

    def fire_src(sx, sb):
      pltpu.async_copy(a_hbm.at[sx.at[pl.ds(0, b)]], sb, gs)

    def fire_dst(dvec):
      dx[...] = dvec
      pltpu.async_copy(a_hbm.at[dx.at[pl.ds(0, b)]], db, gd)

    def drain(buf, g):
      pltpu.make_async_copy(a_hbm.at[pl.ds(0, b)], buf, g).wait()

    def and_write(t, sb):
      for r in range(b):
        def qbody(q, cin):
          for l in range(lanes // 16):
            ls = pl.ds(l * 16, 16)
            sb[r, q, ls] = sb[r, q, ls] & db[r, q, ls]
          return cin
        jax.lax.fori_loop(0, sl, qbody, 0)
      pltpu.sync_copy(sb, o_hbm.at[pl.ds(base + t * b, b)])

    d0 = load_idx(0, sx0)
    fire_src(sx0, sb0)

    def body(u, carry):
      t0 = 2 * u
      t1 = t0 + 1
      dv0 = load_idx(t0, sx0)
      fire_dst(dv0)
      _ = load_idx(t1, sx1)
      drain(sb0, gs)
      fire_src(sx1, sb1)
      drain(db, gd)
      and_write(t0, sb0)
      dv1 = load_idx(t1, sx1)
      fire_dst(dv1)

      @pl.when(t1 + 1 < nb)
      def _():
        load_idx(t1 + 1, sx0)
        drain(sb1, gs)
        fire_src(sx0, sb0)

      @pl.when(t1 + 1 >= nb)
      def _():
        drain(sb1, gs)

      drain(db, gd)
      and_write(t1, sb1)
      return carry

    jax.lax.fori_loop(0, nb // 2, body, 0)

  return gk(A3, pkf)


def _quad_body(nk, adt, v_ref, a_ref, quad_ref, rs_ref, acc_ref):
  j = pl.program_id(1)
  k = pl.program_id(2)
  bk = a_ref.shape[0]
  bj = a_ref.shape[1]

  @pl.when(k == 0)
  def _():
    acc_ref[...] = jnp.zeros_like(acc_ref)

  vk = v_ref[:, pl.ds(k * bk, bk)]
  acc_ref[...] += jax.lax.dot_general(
      vk, a_ref[...], (((1,), (0,)), ((), ())),
      preferred_element_type=adt)

  @pl.when(jnp.logical_and(j == 0, k == 0))
  def _():
    rs_ref[...] = jnp.zeros_like(rs_ref)

  @pl.when(j == 0)
  def _():
    rs_ref[0, 0, :] += jnp.sum(vk.astype(adt), axis=1)

  @pl.when(k == nk - 1)
  def _():
    vj = v_ref[:, pl.ds(j * bj, bj)].astype(adt)
    part = jnp.sum(acc_ref[...] * vj, axis=1)

    @pl.when(j == 0)
    def _():
      quad_ref[0, 0, :] = part

    @pl.when(j > 0)
    def _():
      quad_ref[0, 0, :] += part


def _quad_call(V, A, nchunk, c, bk, bj):
  np_ = A.shape[0]
  nj = np_ // bj
  nk = np_ // bk
  adt = jnp.int32 if V.dtype == jnp.int8 else jnp.float32
  return pl.pallas_call(
      functools.partial(_quad_body, nk, adt),
      grid=(nchunk, nj, nk),
      in_specs=[
          pl.BlockSpec((c, np_), lambda ci, j, k: (ci, 0)),
          pl.BlockSpec((bk, bj), lambda ci, j, k: (k, j)),
      ],
      out_specs=[
          pl.BlockSpec((1, 1, c), lambda ci, j, k: (ci, 0, 0)),
          pl.BlockSpec((1, 1, c), lambda ci, j, k: (ci, 0, 0)),
      ],
      out_shape=[
          jax.ShapeDtypeStruct((nchunk, 1, c), adt),
          jax.ShapeDtypeStruct((nchunk, 1, c), adt),
      ],
      scratch_shapes=[pltpu.VMEM((c, bj), adt)],
      compiler_params=pltpu.CompilerParams(
          dimension_semantics=("parallel", "arbitrary", "arbitrary")),
  )(V, A)


def _segmax_body(ce, pk_ref, nr_ref, y_ref, m_ref):

  @pl.when(pl.program_id(0) == 0)
  def _():
    m_ref[...] = jnp.full_like(m_ref, -jnp.inf)

  def step(e, carry):
    pkv = pk_ref[0, 0, e]
    s = pkv >> 14
    d = pkv & 16383
    nr = nr_ref[0, 0, e]
    row = y_ref[pl.ds(s, 1), :]
    m_ref[pl.ds(d, 1), :] = jnp.maximum(m_ref[pl.ds(d, 1), :], nr * row)
    return carry

  jax.lax.fori_loop(0, ce, step, 0)


def _segmax_call(pk, nr, y, n_out, nchunk, ce):
  d = y.shape[1]
  return pl.pallas_call(
      functools.partial(_segmax_body, ce),
      grid=(nchunk,),
      in_specs=[
          pl.BlockSpec((1, 1, ce), lambda c: (c, 0, 0),
                       memory_space=pltpu.SMEM),
          pl.BlockSpec((1, 1, ce), lambda c: (c, 0, 0),
                       memory_space=pltpu.SMEM),
          pl.BlockSpec(y.shape, lambda c: (0, 0)),
      ],
      out_specs=pl.BlockSpec((n_out, d), lambda c: (0, 0)),
      out_shape=jax.ShapeDtypeStruct((n_out, d), jnp.float32),
      compiler_params=pltpu.CompilerParams(
          dimension_semantics=("arbitrary",)),
  )(pk, nr, y)


def _lin_body(h_ref, w_ref, o_ref):
  o_ref[...] = jax.lax.dot_general(
      h_ref[...], w_ref[...], (((1,), (1,)), ((), ())),
      preferred_element_type=jnp.float32)


def _lin_call(h, w, rb):
  n, _ = h.shape
  dout = w.shape[0]
  return pl.pallas_call(
      _lin_body,
      grid=(n // rb,),
      in_specs=[
          pl.BlockSpec((rb, h.shape[1]), lambda i: (i, 0)),
          pl.BlockSpec(w.shape, lambda i: (0, 0)),
      ],
      out_specs=pl.BlockSpec((rb, dout), lambda i: (i, 0)),
      out_shape=jax.ShapeDtypeStruct((n, dout), jnp.float32),
  )(h, w)


def _lin_relu_body(h_ref, b_ref, w_ref, o_ref):
  a = jnp.maximum(h_ref[...] + b_ref[...], 0.0)
  o_ref[...] = jax.lax.dot_general(
      a, w_ref[...], (((1,), (1,)), ((), ())),
      preferred_element_type=jnp.float32)


def _lin_relu_call(h, b, w, rb):
  n, _ = h.shape
  dout = w.shape[0]
  return pl.pallas_call(
      _lin_relu_body,
      grid=(n // rb,),
      in_specs=[
          pl.BlockSpec((rb, h.shape[1]), lambda i: (i, 0)),
          pl.BlockSpec((1, h.shape[1]), lambda i: (0, 0)),
          pl.BlockSpec(w.shape, lambda i: (0, 0)),
      ],
      out_specs=pl.BlockSpec((rb, dout), lambda i: (i, 0)),
      out_shape=jax.ShapeDtypeStruct((n, dout), jnp.float32),
  )(h, b, w)


def _final_body(h_ref, b_ref, wf_ref, bf_ref, o_ref):
  a = jnp.maximum(h_ref[...] + b_ref[...], 0.0)
  logits = jax.lax.dot_general(
      a, wf_ref[...], (((1,), (1,)), ((), ())),
      preferred_element_type=jnp.float32) + bf_ref[...]
  mx = jnp.max(logits, axis=-1, keepdims=True)
  lse = mx + jnp.log(jnp.sum(jnp.exp(logits - mx), axis=-1, keepdims=True))
  o_ref[...] = logits - lse


def _final_call(h, b, wf, bf, rb):
  n, _ = h.shape
  ncls = wf.shape[0]
  return pl.pallas_call(
      _final_body,
      grid=(n // rb,),
      in_specs=[
          pl.BlockSpec((rb, h.shape[1]), lambda i: (i, 0)),
          pl.BlockSpec((1, h.shape[1]), lambda i: (0, 0)),
          pl.BlockSpec(wf.shape, lambda i: (0, 0)),
          pl.BlockSpec((1, ncls), lambda i: (0, 0)),
      ],
      out_specs=pl.BlockSpec((rb, ncls), lambda i: (i, 0)),
      out_shape=jax.ShapeDtypeStruct((n, ncls), jnp.float32),
  )(h, b, wf, bf)


def kernel(x, edge_index, W0, b0, W1, b1, Wf, bf):
  n, d_in = x.shape
  e = edge_index.shape[1]
  ef = e + n
  np_ = _ceil_to(n, 512)

  diag = jnp.arange(n, dtype=jnp.int32)
  ei0 = edge_index[0].astype(jnp.int32)
  ei1 = edge_index[1].astype(jnp.int32)

  A = jnp.zeros((np_, np_), dtype=jnp.bfloat16)
  A = A.at[ei0, ei1].set(jnp.bfloat16(1))
  A = A.at[ei1, ei0].set(jnp.bfloat16(1))
  A = A.at[diag, diag].set(jnp.bfloat16(1))

  c = 1024
  epad = _ceil_to(ef, c)
  nchunk = epad // c
  zpad = jnp.zeros((epad - ef,), dtype=jnp.int32)
  srcf = jnp.concatenate([ei0, diag, zpad])
  dstf = jnp.concatenate([ei1, diag, zpad])

  half = np_ // 2
  A_pack = jax.lax.bitcast_convert_type(
      A.reshape(np_, half, 2), jnp.int32)
  pkf = (srcf << 14) | dstf
  V_pack = _sc_gather_and(A_pack.reshape(np_, half // 128, 128), pkf)
  V = jax.lax.bitcast_convert_type(
      V_pack.reshape(epad, half), jnp.bfloat16).reshape(epad, np_)

  bk = bj = min(512, np_)
  quad, rs = _quad_call(V, A, nchunk, c, bk, bj)
  quad = quad.reshape(-1)[:ef].astype(jnp.float32)
  rs = rs.reshape(-1)[:ef].astype(jnp.float32)

  inter = 0.5 * (quad + rs)
  ecnt = inter[e:]
  src = srcf[:ef]
  dst = dstf[:ef]
  uni = ecnt[src] + ecnt[dst] - inter
  norm = jnp.where(uni > 0, inter / uni, jnp.float32(0.0))

  ce = 1000 if ef % 1000 == 0 else ef
  echunk = _ceil_to(ef, ce)
  nechunk = echunk // ce
  n_out = n + 8
  pad_e = echunk - ef
  pk = jnp.concatenate([
      (src << 14) | dst,
      jnp.full((pad_e,), n, dtype=jnp.int32),
  ]).reshape(nechunk, 1, ce)
  nrm = jnp.concatenate([norm, jnp.zeros((pad_e,), jnp.float32)])
  nrm = nrm.reshape(nechunk, 1, ce)

  rb = 1000 if n % 1000 == 0 else n

  h = _lin_call(x, W0, rb)
  m = _segmax_call(pk, nrm, h, n_out, nechunk, ce)[:n]
  h = _lin_relu_call(m, b0.reshape(1, -1), W1, rb)
  m = _segmax_call(pk, nrm, h, n_out, nechunk, ce)[:n]
  out = _final_call(m, b1.reshape(1, -1), Wf, bf.reshape(1, -1), rb)
  return out


# speedup vs baseline: 1.3380x; 1.0002x over previous
import functools

import jax
import jax.numpy as jnp
from jax.experimental import pallas as pl
from jax.experimental.pallas import tpu as pltpu
from jax.experimental.pallas import tpu_sc as plsc


def _ceil_to(x, m):
  return ((x + m - 1) // m) * m


def _sc_gather_and(A3, pkf):
  epad, = pkf.shape
  _, sl, lanes = A3.shape
  nw = 32
  rows_w = epad // nw
  b = 8
  nb = rows_w // b
  mesh = plsc.VectorSubcoreMesh(core_axis_name="c", subcore_axis_name="s")

  @functools.partial(
      pl.kernel, mesh=mesh,
      out_type=jax.ShapeDtypeStruct((epad, sl, lanes), A3.dtype),
      scratch_types=[
          pltpu.VMEM((rows_w + 16,), jnp.int32),
          pltpu.VMEM((16,), jnp.int32),
          pltpu.VMEM((16,), jnp.int32),
          pltpu.VMEM((16,), jnp.int32),
          pltpu.VMEM((b, sl, lanes), A3.dtype),
          pltpu.VMEM((b, sl, lanes), A3.dtype),
          pltpu.VMEM((b, sl, lanes), A3.dtype),
          pltpu.SemaphoreType.DMA,
          pltpu.SemaphoreType.DMA,
      ])
  def gk(a_hbm, pk_hbm, o_hbm, pki, sx0, sx1, dx, sb0, sb1, db, gs, gd):
    wid = jax.lax.axis_index("s") * 2 + jax.lax.axis_index("c")
    base = wid * rows_w
    pltpu.sync_copy(pk_hbm.at[pl.ds(base, rows_w)], pki.at[pl.ds(0, rows_w)])

    def load_idx(t, sx):
      v = pki[pl.ds(t * b, 16)]
      sx[...] = v >> 14
      return v & 16383

    def fire_src(sx, sb):
      pltpu.async_copy(a_hbm.at[sx.at[pl.ds(0, b)]], sb, gs)

    def fire_dst(dvec):
      dx[...] = dvec
      pltpu.async_copy(a_hbm.at[dx.at[pl.ds(0, b)]], db, gd)

    def drain(buf, g):
      pltpu.make_async_copy(a_hbm.at[pl.ds(0, b)], buf, g).wait()

    def and_write(t, sb):
      for r in range(b):
        def qbody(q, cin):
          for l in range(lanes // 16):
            ls = pl.ds(l * 16, 16)
            sb[r, q, ls] = sb[r, q, ls] & db[r, q, ls]
          return cin
        jax.lax.fori_loop(0, sl, qbody, 0)
      pltpu.sync_copy(sb, o_hbm.at[pl.ds(base + t * b, b)])

    d0 = load_idx(0, sx0)
    fire_src(sx0, sb0)

    def body(u, carry):
      t0 = 2 * u
      t1 = t0 + 1
      dv0 = load_idx(t0, sx0)
      fire_dst(dv0)
      _ = load_idx(t1, sx1)
      drain(sb0, gs)
      fire_src(sx1, sb1)
      drain(db, gd)
      and_write(t0, sb0)
      dv1 = load_idx(t1, sx1)
      fire_dst(dv1)

      @pl.when(t1 + 1 < nb)
      def _():
        load_idx(t1 + 1, sx0)
        drain(sb1, gs)
        fire_src(sx0, sb0)

      @pl.when(t1 + 1 >= nb)
      def _():
        drain(sb1, gs)

      drain(db, gd)
      and_write(t1, sb1)
      return carry

    jax.lax.fori_loop(0, nb // 2, body, 0)

  return gk(A3, pkf)


def _quad_body(nk, adt, v_ref, a_ref, quad_ref, rs_ref, acc_ref):
  j = pl.program_id(1)
  k = pl.program_id(2)
  bk = a_ref.shape[0]
  bj = a_ref.shape[1]

  @pl.when(k == 0)
  def _():
    acc_ref[...] = jnp.zeros_like(acc_ref)

  vk = v_ref[:, pl.ds(k * bk, bk)]
  acc_ref[...] += jax.lax.dot_general(
      vk, a_ref[...], (((1,), (0,)), ((), ())),
      preferred_element_type=adt)

  @pl.when(jnp.logical_and(j == 0, k == 0))
  def _():
    rs_ref[...] = jnp.zeros_like(rs_ref)

  @pl.when(j == 0)
  def _():
    rs_ref[0, 0, :] += jnp.sum(vk.astype(adt), axis=1)

  @pl.when(k == nk - 1)
  def _():
    vj = v_ref[:, pl.ds(j * bj, bj)].astype(adt)
    part = jnp.sum(acc_ref[...] * vj, axis=1)

    @pl.when(j == 0)
    def _():
      quad_ref[0, 0, :] = part

    @pl.when(j > 0)
    def _():
      quad_ref[0, 0, :] += part


def _quad_call(V, A, nchunk, c, bk, bj):
  np_ = A.shape[0]
  nj = np_ // bj
  nk = np_ // bk
  adt = jnp.float32
  return pl.pallas_call(
      functools.partial(_quad_body, nk, adt),
      grid=(nchunk, nj, nk),
      in_specs=[
          pl.BlockSpec((c, np_), lambda ci, j, k: (ci, 0)),
          pl.BlockSpec((bk, bj), lambda ci, j, k: (k, j)),
      ],
      out_specs=[
          pl.BlockSpec((1, 1, c), lambda ci, j, k: (ci, 0, 0)),
          pl.BlockSpec((1, 1, c), lambda ci, j, k: (ci, 0, 0)),
      ],
      out_shape=[
          jax.ShapeDtypeStruct((nchunk, 1, c), adt),
          jax.ShapeDtypeStruct((nchunk, 1, c), adt),
      ],
      scratch_shapes=[pltpu.VMEM((c, bj), adt)],
      compiler_params=pltpu.CompilerParams(
          dimension_semantics=("parallel", "arbitrary", "arbitrary")),
  )(V, A)


def _segmax_body(ce, pk_ref, nr_ref, y_ref, m_ref):

  @pl.when(pl.program_id(0) == 0)
  def _():
    m_ref[...] = jnp.full_like(m_ref, -jnp.inf)

  def step(e, carry):
    pkv = pk_ref[0, 0, e]
    s = pkv >> 14
    d = pkv & 16383
    nr = nr_ref[0, 0, e]
    row = y_ref[pl.ds(s, 1), :]
    m_ref[pl.ds(d, 1), :] = jnp.maximum(m_ref[pl.ds(d, 1), :], nr * row)
    return carry

  jax.lax.fori_loop(0, ce, step, 0)


def _segmax_call(pk, nr, y, n_out, nchunk, ce):
  d = y.shape[1]
  return pl.pallas_call(
      functools.partial(_segmax_body, ce),
      grid=(nchunk,),
      in_specs=[
          pl.BlockSpec((1, 1, ce), lambda c: (c, 0, 0),
                       memory_space=pltpu.SMEM),
          pl.BlockSpec((1, 1, ce), lambda c: (c, 0, 0),
                       memory_space=pltpu.SMEM),
          pl.BlockSpec(y.shape, lambda c: (0, 0)),
      ],
      out_specs=pl.BlockSpec((n_out, d), lambda c: (0, 0)),
      out_shape=jax.ShapeDtypeStruct((n_out, d), jnp.float32),
      compiler_params=pltpu.CompilerParams(
          dimension_semantics=("arbitrary",)),
  )(pk, nr, y)


def _lin_body(h_ref, w_ref, o_ref):
  o_ref[...] = jax.lax.dot_general(
      h_ref[...], w_ref[...], (((1,), (1,)), ((), ())),
      preferred_element_type=jnp.float32)


def _lin_call(h, w, rb):
  n, _ = h.shape
  dout = w.shape[0]
  return pl.pallas_call(
      _lin_body,
      grid=(n // rb,),
      in_specs=[
          pl.BlockSpec((rb, h.shape[1]), lambda i: (i, 0)),
          pl.BlockSpec(w.shape, lambda i: (0, 0)),
      ],
      out_specs=pl.BlockSpec((rb, dout), lambda i: (i, 0)),
      out_shape=jax.ShapeDtypeStruct((n, dout), jnp.float32),
  )(h, w)


def _lin_relu_body(h_ref, b_ref, w_ref, o_ref):
  a = jnp.maximum(h_ref[...] + b_ref[...], 0.0)
  o_ref[...] = jax.lax.dot_general(
      a, w_ref[...], (((1,), (1,)), ((), ())),
      preferred_element_type=jnp.float32)


def _lin_relu_call(h, b, w, rb):
  n, _ = h.shape
  dout = w.shape[0]
  return pl.pallas_call(
      _lin_relu_body,
      grid=(n // rb,),
      in_specs=[
          pl.BlockSpec((rb, h.shape[1]), lambda i: (i, 0)),
          pl.BlockSpec((1, h.shape[1]), lambda i: (0, 0)),
          pl.BlockSpec(w.shape, lambda i: (0, 0)),
      ],
      out_specs=pl.BlockSpec((rb, dout), lambda i: (i, 0)),
      out_shape=jax.ShapeDtypeStruct((n, dout), jnp.float32),
  )(h, b, w)


def _final_body(h_ref, b_ref, wf_ref, bf_ref, o_ref):
  a = jnp.maximum(h_ref[...] + b_ref[...], 0.0)
  logits = jax.lax.dot_general(
      a, wf_ref[...], (((1,), (1,)), ((), ())),
      preferred_element_type=jnp.float32) + bf_ref[...]
  mx = jnp.max(logits, axis=-1, keepdims=True)
  lse = mx + jnp.log(jnp.sum(jnp.exp(logits - mx), axis=-1, keepdims=True))
  o_ref[...] = logits - lse


def _final_call(h, b, wf, bf, rb):
  n, _ = h.shape
  ncls = wf.shape[0]
  return pl.pallas_call(
      _final_body,
      grid=(n // rb,),
      in_specs=[
          pl.BlockSpec((rb, h.shape[1]), lambda i: (i, 0)),
          pl.BlockSpec((1, h.shape[1]), lambda i: (0, 0)),
          pl.BlockSpec(wf.shape, lambda i: (0, 0)),
          pl.BlockSpec((1, ncls), lambda i: (0, 0)),
      ],
      out_specs=pl.BlockSpec((rb, ncls), lambda i: (i, 0)),
      out_shape=jax.ShapeDtypeStruct((n, ncls), jnp.float32),
  )(h, b, wf, bf)


def kernel(x, edge_index, W0, b0, W1, b1, Wf, bf):
  n, d_in = x.shape
  e = edge_index.shape[1]
  ef = e + n
  np_ = _ceil_to(n, 512)

  diag = jnp.arange(n, dtype=jnp.int32)
  ei0 = edge_index[0].astype(jnp.int32)
  ei1 = edge_index[1].astype(jnp.int32)

  A = jnp.zeros((np_, np_), dtype=jnp.bfloat16)
  A = A.at[ei0, ei1].set(jnp.bfloat16(1))
  A = A.at[ei1, ei0].set(jnp.bfloat16(1))
  A = A.at[diag, diag].set(jnp.bfloat16(1))

  c = 1024
  epad = _ceil_to(ef, c)
  nchunk = epad // c
  zpad = jnp.zeros((epad - ef,), dtype=jnp.int32)
  srcf = jnp.concatenate([ei0, diag, zpad])
  dstf = jnp.concatenate([ei1, diag, zpad])

  half = np_ // 2
  A_pack = jax.lax.bitcast_convert_type(
      A.reshape(np_, half, 2), jnp.int32)
  pkf = (srcf << 14) | dstf
  V_pack = _sc_gather_and(A_pack.reshape(np_, half // 128, 128), pkf)
  V = jax.lax.bitcast_convert_type(
      V_pack.reshape(epad, half), jnp.bfloat16).reshape(epad, np_)

  bk = bj = min(512, np_)
  quad, rs = _quad_call(V, A, nchunk, c, bk, bj)
  quad = quad.reshape(-1)[:ef].astype(jnp.float32)
  rs = rs.reshape(-1)[:ef].astype(jnp.float32)

  inter = 0.5 * (quad + rs)
  ecnt = inter[e:]
  src = srcf[:ef]
  dst = dstf[:ef]
  uni = ecnt[src] + ecnt[dst] - inter
  norm = jnp.where(uni > 0, inter / uni, jnp.float32(0.0))

  ce = 1000 if ef % 1000 == 0 else ef
  echunk = _ceil_to(ef, ce)
  nechunk = echunk // ce
  n_out = n + 8
  pad_e = echunk - ef
  pk = jnp.concatenate([
      (src << 14) | dst,
      jnp.full((pad_e,), n, dtype=jnp.int32),
  ]).reshape(nechunk, 1, ce)
  nrm = jnp.concatenate([norm, jnp.zeros((pad_e,), jnp.float32)])
  nrm = nrm.reshape(nechunk, 1, ce)

  rb = 1000 if n % 1000 == 0 else n

  h = _lin_call(x, W0, rb)
  m = _segmax_call(pk, nrm, h, n_out, nechunk, ce)[:n]
  h = _lin_relu_call(m, b0.reshape(1, -1), W1, rb)
  m = _segmax_call(pk, nrm, h, n_out, nechunk, ce)[:n]
  out = _final_call(m, b1.reshape(1, -1), Wf, bf.reshape(1, -1), rb)
  return out
